# Initial kernel scaffold; baseline (speedup 1.0000x reference)
import jax
import jax.numpy as jnp
from jax.experimental import pallas as pl


def kernel(elems, atomic_energy, scales, shifts, elem_lookup):
    raise NotImplementedError("write your pallas kernel here")


# speedup vs baseline: 65.8932x; 65.8932x over previous
import dataclasses

import jax
import jax.numpy as jnp
from jax import lax
from jax.experimental import pallas as pl
from jax.experimental.pallas import tpu as pltpu
from jax.experimental.pallas import tpu_sc as plsc

N = 100000
NC, NS, L = 2, 16, 16
NW = NC * NS
VECS = N // L
MAIN_VECS = VECS // NW
CHUNK = MAIN_VECS * L
TAIL_VECS = VECS - MAIN_VECS * NW
TAIL_BASE = NW * CHUNK
TBL = 112

_mesh = plsc.VectorSubcoreMesh(
    core_axis_name="c", subcore_axis_name="s", num_cores=NC, num_subcores=NS
)

_cp = pltpu.CompilerParams()
if "needs_layout_passes" in pltpu.CompilerParams.__dataclass_fields__:
    _cp = dataclasses.replace(_cp, needs_layout_passes=False)


def _body(
    elems_hbm,
    energy_hbm,
    scales_hbm,
    shifts_hbm,
    lookup_hbm,
    out_hbm,
    elems_v,
    energy_v,
    out_v,
    lookup_v,
    scale_t,
    shift_t,
    cs_v,
    cb_v,
    te_i,
    te_f,
    to_f,
):
    wid = lax.axis_index("c") * NS + lax.axis_index("s")
    base = pl.multiple_of(wid * CHUNK, 16)

    lookup_v[pl.ds(TBL - L, L)] = jnp.zeros((L,), jnp.int32)
    pltpu.sync_copy(lookup_hbm, lookup_v.at[pl.ds(0, 100)])
    pltpu.sync_copy(scales_hbm, scale_t.at[pl.ds(0, 99)])
    pltpu.sync_copy(shifts_hbm, shift_t.at[pl.ds(0, 99)])

    for e0 in range(0, TBL, L):
        lv = lookup_v[pl.ds(e0, L)]
        cs_v[pl.ds(e0, L)] = plsc.load_gather(scale_t, [lv])
        cb_v[pl.ds(e0, L)] = plsc.load_gather(shift_t, [lv])

    pltpu.sync_copy(elems_hbm.at[pl.ds(base, CHUNK)], elems_v)
    pltpu.sync_copy(energy_hbm.at[pl.ds(base, CHUNK)], energy_v)

    @pl.loop(0, MAIN_VECS)
    def _(i):
        o = i * L
        ev = elems_v[pl.ds(o, L)]
        en = energy_v[pl.ds(o, L)]
        out_v[pl.ds(o, L)] = (
            en * plsc.load_gather(cs_v, [ev]) + plsc.load_gather(cb_v, [ev])
        )

    pltpu.sync_copy(out_v, out_hbm.at[pl.ds(base, CHUNK)])

    @pl.when(wid < TAIL_VECS)
    def _():
        tb = pl.multiple_of(TAIL_BASE + wid * L, 16)
        pltpu.sync_copy(elems_hbm.at[pl.ds(tb, L)], te_i)
        pltpu.sync_copy(energy_hbm.at[pl.ds(tb, L)], te_f)
        ev = te_i[...]
        to_f[...] = (
            te_f[...] * plsc.load_gather(cs_v, [ev])
            + plsc.load_gather(cb_v, [ev])
        )
        pltpu.sync_copy(to_f, out_hbm.at[pl.ds(tb, L)])


def kernel(elems, atomic_energy, scales, shifts, elem_lookup):
    k = pl.kernel(
        _body,
        out_type=jax.ShapeDtypeStruct((N,), jnp.float32),
        mesh=_mesh,
        compiler_params=_cp,
        scratch_types=[
            pltpu.VMEM((CHUNK,), jnp.int32),
            pltpu.VMEM((CHUNK,), jnp.float32),
            pltpu.VMEM((CHUNK,), jnp.float32),
            pltpu.VMEM((TBL,), jnp.int32),
            pltpu.VMEM((TBL,), jnp.float32),
            pltpu.VMEM((TBL,), jnp.float32),
            pltpu.VMEM((TBL,), jnp.float32),
            pltpu.VMEM((TBL,), jnp.float32),
            pltpu.VMEM((L,), jnp.int32),
            pltpu.VMEM((L,), jnp.float32),
            pltpu.VMEM((L,), jnp.float32),
        ],
    )
    return k(elems, atomic_energy, scales, shifts, elem_lookup)


# speedup vs baseline: 72.1561x; 1.0950x over previous
import dataclasses

import jax
import jax.numpy as jnp
from jax import lax
from jax.experimental import pallas as pl
from jax.experimental.pallas import tpu as pltpu
from jax.experimental.pallas import tpu_sc as plsc

N = 100000
NC, NS, L = 2, 16, 16
NW = NC * NS
VECS = N // L
MAIN_VECS = VECS // NW
CHUNK = MAIN_VECS * L
TAIL_VECS = VECS - MAIN_VECS * NW
TAIL_BASE = NW * CHUNK
TBL = 112

_mesh = plsc.VectorSubcoreMesh(
    core_axis_name="c", subcore_axis_name="s", num_cores=NC, num_subcores=NS
)

_cp = pltpu.CompilerParams()
if "needs_layout_passes" in pltpu.CompilerParams.__dataclass_fields__:
    _cp = dataclasses.replace(_cp, needs_layout_passes=False)


def _body(
    elems_hbm,
    energy_hbm,
    scales_hbm,
    shifts_hbm,
    lookup_hbm,
    out_hbm,
    elems_v,
    energy_v,
    out_v,
    lookup_v,
    scale_t,
    shift_t,
    cs_v,
    cb_v,
    te_i,
    te_f,
    to_f,
    sem_tbl,
    sem_in,
    sem_tail,
    sem_out,
):
    wid = lax.axis_index("c") * NS + lax.axis_index("s")
    base = pl.multiple_of(wid * CHUNK, 16)
    is_tail = wid < TAIL_VECS
    tb = pl.multiple_of(TAIL_BASE + wid * L, 16)

    lookup_v[pl.ds(TBL - L, L)] = jnp.zeros((L,), jnp.int32)

    pltpu.async_copy(lookup_hbm, lookup_v.at[pl.ds(0, 100)], sem_tbl)
    pltpu.async_copy(scales_hbm, scale_t.at[pl.ds(0, 99)], sem_tbl)
    pltpu.async_copy(shifts_hbm, shift_t.at[pl.ds(0, 99)], sem_tbl)
    pltpu.async_copy(elems_hbm.at[pl.ds(base, CHUNK)], elems_v, sem_in)
    pltpu.async_copy(energy_hbm.at[pl.ds(base, CHUNK)], energy_v, sem_in)

    @pl.when(is_tail)
    def _():
        pltpu.async_copy(elems_hbm.at[pl.ds(tb, L)], te_i, sem_tail)
        pltpu.async_copy(energy_hbm.at[pl.ds(tb, L)], te_f, sem_tail)

    pltpu.make_async_copy(lookup_hbm, lookup_v.at[pl.ds(0, 100)], sem_tbl).wait()
    pltpu.make_async_copy(scales_hbm, scale_t.at[pl.ds(0, 99)], sem_tbl).wait()
    pltpu.make_async_copy(shifts_hbm, shift_t.at[pl.ds(0, 99)], sem_tbl).wait()
    for e0 in range(0, TBL, L):
        lv = lookup_v[pl.ds(e0, L)]
        cs_v[pl.ds(e0, L)] = plsc.load_gather(scale_t, [lv])
        cb_v[pl.ds(e0, L)] = plsc.load_gather(shift_t, [lv])

    pltpu.make_async_copy(elems_hbm.at[pl.ds(base, CHUNK)], elems_v, sem_in).wait()
    pltpu.make_async_copy(energy_hbm.at[pl.ds(base, CHUNK)], energy_v, sem_in).wait()

    @pl.loop(0, MAIN_VECS, unroll=5)
    def _(i):
        o = i * L
        ev = elems_v[pl.ds(o, L)]
        en = energy_v[pl.ds(o, L)]
        out_v[pl.ds(o, L)] = (
            en * plsc.load_gather(cs_v, [ev]) + plsc.load_gather(cb_v, [ev])
        )

    pltpu.async_copy(out_v, out_hbm.at[pl.ds(base, CHUNK)], sem_out)

    @pl.when(is_tail)
    def _():
        pltpu.make_async_copy(elems_hbm.at[pl.ds(tb, L)], te_i, sem_tail).wait()
        pltpu.make_async_copy(energy_hbm.at[pl.ds(tb, L)], te_f, sem_tail).wait()
        ev = te_i[...]
        to_f[...] = (
            te_f[...] * plsc.load_gather(cs_v, [ev])
            + plsc.load_gather(cb_v, [ev])
        )
        pltpu.sync_copy(to_f, out_hbm.at[pl.ds(tb, L)])

    pltpu.make_async_copy(out_v, out_hbm.at[pl.ds(base, CHUNK)], sem_out).wait()


def kernel(elems, atomic_energy, scales, shifts, elem_lookup):
    k = pl.kernel(
        _body,
        out_type=jax.ShapeDtypeStruct((N,), jnp.float32),
        mesh=_mesh,
        compiler_params=_cp,
        scratch_types=[
            pltpu.VMEM((CHUNK,), jnp.int32),
            pltpu.VMEM((CHUNK,), jnp.float32),
            pltpu.VMEM((CHUNK,), jnp.float32),
            pltpu.VMEM((TBL,), jnp.int32),
            pltpu.VMEM((TBL,), jnp.float32),
            pltpu.VMEM((TBL,), jnp.float32),
            pltpu.VMEM((TBL,), jnp.float32),
            pltpu.VMEM((TBL,), jnp.float32),
            pltpu.VMEM((L,), jnp.int32),
            pltpu.VMEM((L,), jnp.float32),
            pltpu.VMEM((L,), jnp.float32),
            pltpu.SemaphoreType.DMA,
            pltpu.SemaphoreType.DMA,
            pltpu.SemaphoreType.DMA,
            pltpu.SemaphoreType.DMA,
        ],
    )
    return k(elems, atomic_energy, scales, shifts, elem_lookup)
